# baseline (speedup 1.0000x reference)
import jax
import jax.numpy as jnp
from jax.experimental import pallas as pl
from jax.experimental.pallas import tpu as pltpu

_LANES = 128
_NCHUNK = 8
_WIN = 8


def _dma_kernel(idx_ref, a_ref, o_ref, vscr, *sems):
    rows = a_ref.shape[0]
    chunk = rows // _NCHUNK
    copies = []
    for c in range(_NCHUNK):
        cp = pltpu.make_async_copy(
            a_ref.at[pl.ds(c * chunk, chunk), :],
            o_ref.at[pl.ds(c * chunk, chunk), :],
            sems[c],
        )
        cp.start()
        copies.append(cp)
    for cp in copies:
        cp.wait()

    idx = idx_ref[0]
    r0 = jnp.minimum(idx // _LANES, rows - _WIN)
    win_in = pltpu.make_async_copy(
        o_ref.at[pl.ds(r0, _WIN), :], vscr, sems[_NCHUNK])
    win_in.start()
    win_in.wait()
    rr = jax.lax.broadcasted_iota(jnp.int32, (_WIN, _LANES), 0)
    cc = jax.lax.broadcasted_iota(jnp.int32, (_WIN, _LANES), 1)
    flat = (r0 + rr) * _LANES + cc
    mask = jnp.logical_or(flat == idx, flat == idx + 1)
    vscr[...] = jnp.where(mask, jnp.float32(0), vscr[...])
    win_out = pltpu.make_async_copy(
        vscr, o_ref.at[pl.ds(r0, _WIN), :], sems[_NCHUNK])
    win_out.start()
    win_out.wait()


def kernel(a, idx):
    n = a.shape[0]
    rows = n // _LANES
    idx32 = idx.astype(jnp.int32)
    a2 = a.reshape(rows, _LANES)
    out = pl.pallas_call(
        _dma_kernel,
        out_shape=jax.ShapeDtypeStruct((rows, _LANES), a.dtype),
        in_specs=[
            pl.BlockSpec(memory_space=pltpu.SMEM),
            pl.BlockSpec(memory_space=pltpu.MemorySpace.HBM),
        ],
        out_specs=pl.BlockSpec(memory_space=pltpu.MemorySpace.HBM),
        scratch_shapes=[pltpu.VMEM((_WIN, _LANES), jnp.float32)]
        + [pltpu.SemaphoreType.DMA] * (_NCHUNK + 1),
    )(idx32, a2)
    return out.reshape(n)


# speedup vs baseline: 46.3620x; 46.3620x over previous
import jax
import jax.numpy as jnp
from jax.experimental import pallas as pl
from jax.experimental.pallas import tpu as pltpu

_LANES = 128
_BLOCK_ROWS = 8192
_BLOCK = _BLOCK_ROWS * _LANES


def _copy_zero_kernel(idx_ref, a_ref, o_ref):
    i = pl.program_id(0)
    idx = idx_ref[0]
    base = i * _BLOCK

    contains = jnp.logical_and(idx + 1 >= base, idx < base + _BLOCK)

    @pl.when(jnp.logical_not(contains))
    def _plain():
        o_ref[...] = a_ref[...]

    @pl.when(contains)
    def _masked():
        rows = jax.lax.broadcasted_iota(jnp.int32, (_BLOCK_ROWS, _LANES), 0)
        cols = jax.lax.broadcasted_iota(jnp.int32, (_BLOCK_ROWS, _LANES), 1)
        flat = base + rows * _LANES + cols
        mask = jnp.logical_or(flat == idx, flat == idx + 1)
        o_ref[...] = jnp.where(mask, jnp.float32(0), a_ref[...])


def kernel(a, idx):
    n = a.shape[0]
    rows = n // _LANES
    grid = rows // _BLOCK_ROWS
    idx32 = idx.astype(jnp.int32)
    a2 = a.reshape(rows, _LANES)
    out = pl.pallas_call(
        _copy_zero_kernel,
        out_shape=jax.ShapeDtypeStruct((rows, _LANES), a.dtype),
        grid=(grid,),
        in_specs=[
            pl.BlockSpec(memory_space=pltpu.SMEM),
            pl.BlockSpec((_BLOCK_ROWS, _LANES), lambda i: (i, 0)),
        ],
        out_specs=pl.BlockSpec((_BLOCK_ROWS, _LANES), lambda i: (i, 0)),
    )(idx32, a2)
    return out.reshape(n)


# speedup vs baseline: 48.1467x; 1.0385x over previous
import jax
import jax.numpy as jnp
from jax.experimental import pallas as pl
from jax.experimental.pallas import tpu as pltpu

_LANES = 128
_BLOCK_ROWS = 16384
_BLOCK = _BLOCK_ROWS * _LANES


def _copy_zero_kernel(idx_ref, a_ref, o_ref):
    i = pl.program_id(0)
    idx = idx_ref[0]
    base = i * _BLOCK

    contains = jnp.logical_and(idx + 1 >= base, idx < base + _BLOCK)

    @pl.when(jnp.logical_not(contains))
    def _plain():
        o_ref[...] = a_ref[...]

    @pl.when(contains)
    def _masked():
        rows = jax.lax.broadcasted_iota(jnp.int32, (_BLOCK_ROWS, _LANES), 0)
        cols = jax.lax.broadcasted_iota(jnp.int32, (_BLOCK_ROWS, _LANES), 1)
        flat = base + rows * _LANES + cols
        mask = jnp.logical_or(flat == idx, flat == idx + 1)
        o_ref[...] = jnp.where(mask, jnp.float32(0), a_ref[...])


def kernel(a, idx):
    n = a.shape[0]
    rows = n // _LANES
    grid = rows // _BLOCK_ROWS
    idx32 = idx.astype(jnp.int32)
    a2 = a.reshape(rows, _LANES)
    out = pl.pallas_call(
        _copy_zero_kernel,
        out_shape=jax.ShapeDtypeStruct((rows, _LANES), a.dtype),
        grid=(grid,),
        in_specs=[
            pl.BlockSpec(memory_space=pltpu.SMEM),
            pl.BlockSpec((_BLOCK_ROWS, _LANES), lambda i: (i, 0)),
        ],
        out_specs=pl.BlockSpec((_BLOCK_ROWS, _LANES), lambda i: (i, 0)),
    )(idx32, a2)
    return out.reshape(n)
